# initial kernel scaffold (unmeasured)
import jax
import jax.numpy as jnp
from jax import lax
from jax.experimental import pallas as pl
from jax.experimental.pallas import tpu as pltpu


def kernel(
    u,
):
    def body(*refs):
        pass

    out_shape = jax.ShapeDtypeStruct(..., jnp.float32)
    return pl.pallas_call(body, out_shape=out_shape)(...)


# baseline (device time: 7351 ns/iter reference)
import jax
import jax.numpy as jnp
from jax import lax
from jax.experimental import pallas as pl
from jax.experimental.pallas import tpu as pltpu

N = 16
G = 32


def kernel(u):
    def body(u_ref, out_ref, send_buf, recv_buf, send_sems, recv_sems):
        mx = lax.axis_index("x")
        my = lax.axis_index("y")
        mz = lax.axis_index("z")

        nbrs = [(1 - mx, my, mz), (mx, 1 - my, mz), (mx, my, 1 - mz)]

        barrier = pltpu.get_barrier_semaphore()
        for nbr in nbrs:
            pl.semaphore_signal(
                barrier, inc=1, device_id=nbr,
                device_id_type=pl.DeviceIdType.MESH,
            )
        pl.semaphore_wait(barrier, 3)

        u_val = u_ref[:, :, :]

        send_buf[0, :, :] = jnp.where(mx == 0, u_val[N - 1, :, :], u_val[0, :, :])
        send_buf[1, :, :] = jnp.where(my == 0, u_val[:, N - 1, :], u_val[:, 0, :])
        send_buf[2, :, :] = jnp.where(mz == 0, u_val[:, :, N - 1], u_val[:, :, 0])

        rdmas = []
        for a, nbr in enumerate(nbrs):
            rdma = pltpu.make_async_remote_copy(
                src_ref=send_buf.at[a],
                dst_ref=recv_buf.at[a],
                send_sem=send_sems.at[a],
                recv_sem=recv_sems.at[a],
                device_id=nbr,
                device_id_type=pl.DeviceIdType.MESH,
            )
            rdma.start()
            rdmas.append(rdma)
        for rdma in rdmas:
            rdma.wait()

        halo_x = recv_buf[0, :, :]
        halo_y = recv_buf[1, :, :]
        halo_z = recv_buf[2, :, :]

        zero = jnp.zeros((N, N), jnp.float32)
        lo_x = jnp.where(mx == 0, zero, halo_x)
        hi_x = jnp.where(mx == 0, halo_x, zero)
        lo_y = jnp.where(my == 0, zero, halo_y)
        hi_y = jnp.where(my == 0, halo_y, zero)
        lo_z = jnp.where(mz == 0, zero, halo_z)
        hi_z = jnp.where(mz == 0, halo_z, zero)

        um_x = jnp.concatenate([lo_x[None, :, :], u_val[:-1, :, :]], axis=0)
        up_x = jnp.concatenate([u_val[1:, :, :], hi_x[None, :, :]], axis=0)
        um_y = jnp.concatenate([lo_y[:, None, :], u_val[:, :-1, :]], axis=1)
        up_y = jnp.concatenate([u_val[:, 1:, :], hi_y[:, None, :]], axis=1)
        um_z = jnp.concatenate([lo_z[:, :, None], u_val[:, :, :-1]], axis=2)
        up_z = jnp.concatenate([u_val[:, :, 1:], hi_z[:, :, None]], axis=2)

        v = um_x + up_x + um_y + up_y + um_z + up_z - 6.0 * u_val

        gx = lax.broadcasted_iota(jnp.int32, (N, N, N), 0) + mx * N
        gy = lax.broadcasted_iota(jnp.int32, (N, N, N), 1) + my * N
        gz = lax.broadcasted_iota(jnp.int32, (N, N, N), 2) + mz * N
        interior = (
            (gx > 0) & (gx < G - 1)
            & (gy > 0) & (gy < G - 1)
            & (gz > 0) & (gz < G - 1)
        )
        out_ref[:, :, :] = jnp.where(interior, v, 0.0)

    return pl.pallas_call(
        body,
        out_shape=jax.ShapeDtypeStruct((N, N, N), jnp.float32),
        in_specs=[pl.BlockSpec(memory_space=pltpu.VMEM)],
        out_specs=pl.BlockSpec(memory_space=pltpu.VMEM),
        scratch_shapes=[
            pltpu.VMEM((3, N, N), jnp.float32),
            pltpu.VMEM((3, N, N), jnp.float32),
            pltpu.SemaphoreType.DMA((3,)),
            pltpu.SemaphoreType.DMA((3,)),
        ],
        compiler_params=pltpu.CompilerParams(collective_id=0),
    )(u)


# device time: 7228 ns/iter; 1.0170x vs baseline; 1.0170x over previous
import jax
import jax.numpy as jnp
from jax import lax
from jax.experimental import pallas as pl
from jax.experimental.pallas import tpu as pltpu

N = 16
G = 32


def kernel(u):
    def body(u_ref, out_ref, send_buf, recv_buf, send_sems, recv_sems):
        mx = lax.axis_index("x")
        my = lax.axis_index("y")
        mz = lax.axis_index("z")

        nbrs = [(1 - mx, my, mz), (mx, 1 - my, mz), (mx, my, 1 - mz)]

        u_val = u_ref[:, :, :]

        send_buf[0, :, :] = jnp.where(mx == 0, u_val[N - 1, :, :], u_val[0, :, :])
        send_buf[1, :, :] = jnp.where(my == 0, u_val[:, N - 1, :], u_val[:, 0, :])
        send_buf[2, :, :] = jnp.where(mz == 0, u_val[:, :, N - 1], u_val[:, :, 0])

        barrier = pltpu.get_barrier_semaphore()
        for nbr in nbrs:
            pl.semaphore_signal(
                barrier, inc=1, device_id=nbr,
                device_id_type=pl.DeviceIdType.MESH,
            )
        pl.semaphore_wait(barrier, 3)

        rdmas = []
        for a, nbr in enumerate(nbrs):
            rdma = pltpu.make_async_remote_copy(
                src_ref=send_buf.at[a],
                dst_ref=recv_buf.at[a],
                send_sem=send_sems.at[a],
                recv_sem=recv_sems.at[a],
                device_id=nbr,
                device_id_type=pl.DeviceIdType.MESH,
            )
            rdma.start()
            rdmas.append(rdma)

        zplane = jnp.zeros((1, N, N), jnp.float32)
        v = (
            jnp.concatenate([zplane, u_val[:-1, :, :]], axis=0)
            + jnp.concatenate([u_val[1:, :, :], zplane], axis=0)
            + jnp.concatenate([zplane.reshape(N, 1, N), u_val[:, :-1, :]], axis=1)
            + jnp.concatenate([u_val[:, 1:, :], zplane.reshape(N, 1, N)], axis=1)
            + jnp.concatenate([zplane.reshape(N, N, 1), u_val[:, :, :-1]], axis=2)
            + jnp.concatenate([u_val[:, :, 1:], zplane.reshape(N, N, 1)], axis=2)
            - 6.0 * u_val
        )

        lx = lax.broadcasted_iota(jnp.int32, (N, N, N), 0)
        ly = lax.broadcasted_iota(jnp.int32, (N, N, N), 1)
        lz = lax.broadcasted_iota(jnp.int32, (N, N, N), 2)

        gx, gy, gz = lx + mx * N, ly + my * N, lz + mz * N
        interior = (
            (gx > 0) & (gx < G - 1)
            & (gy > 0) & (gy < G - 1)
            & (gz > 0) & (gz < G - 1)
        )

        for rdma in rdmas:
            rdma.wait()

        v = v + jnp.where(lx == (1 - mx) * (N - 1), recv_buf[0, :, :][None, :, :], 0.0)
        v = v + jnp.where(ly == (1 - my) * (N - 1), recv_buf[1, :, :][:, None, :], 0.0)
        v = v + jnp.where(lz == (1 - mz) * (N - 1), recv_buf[2, :, :][:, :, None], 0.0)

        out_ref[:, :, :] = jnp.where(interior, v, 0.0)

    return pl.pallas_call(
        body,
        out_shape=jax.ShapeDtypeStruct((N, N, N), jnp.float32),
        in_specs=[pl.BlockSpec(memory_space=pltpu.VMEM)],
        out_specs=pl.BlockSpec(memory_space=pltpu.VMEM),
        scratch_shapes=[
            pltpu.VMEM((3, N, N), jnp.float32),
            pltpu.VMEM((3, N, N), jnp.float32),
            pltpu.SemaphoreType.DMA((3,)),
            pltpu.SemaphoreType.DMA((3,)),
        ],
        compiler_params=pltpu.CompilerParams(collective_id=0),
    )(u)
